# Initial kernel scaffold; baseline (speedup 1.0000x reference)
import jax
import jax.numpy as jnp
from jax.experimental import pallas as pl


def kernel(x_in, pos_in, batch_in, in_index, out_index, W1, W2, W3, b3):
    raise NotImplementedError("write your pallas kernel here")


# speedup vs baseline: 16.4756x; 16.4756x over previous
import functools

import jax
import jax.numpy as jnp
from jax import lax
from jax.experimental import pallas as pl
from jax.experimental.pallas import tpu as pltpu
from jax.experimental.pallas import tpu_sc as plsc

N = 10000
E = 320000
C_IN = 16
C_MID = 64
C_OUT = 64
TBL_W = 32
Y_W = 80
CHUNK = 128
NCHUNK = E // CHUNK
NW = 32
CHUNK_ITERS = (NCHUNK + NW - 1) // NW
ROWS_PER_TILE = N // 16

_mesh = plsc.VectorSubcoreMesh(core_axis_name="c", subcore_axis_name="s")


@functools.partial(
    pl.kernel,
    out_type=(
        jax.ShapeDtypeStruct((E, TBL_W), jnp.float32),
        jax.ShapeDtypeStruct((E, TBL_W), jnp.float32),
    ),
    mesh=_mesh,
    compiler_params=pltpu.CompilerParams(use_tc_tiling_on_sc=False),
    scratch_types=[
        pltpu.VMEM((CHUNK,), jnp.int32),
        pltpu.VMEM((CHUNK, TBL_W), jnp.float32),
        pltpu.SemaphoreType.DMA,
    ],
)
def _sc_gather(table_hbm, ii_hbm, oi_hbm, gii_hbm, goi_hbm, idx_v, rows_v, sem):
    wid = lax.axis_index("s") * 2 + lax.axis_index("c")

    def body(j, carry):
        cid = wid + j * NW

        @pl.when(cid < NCHUNK)
        def _():
            base = cid * CHUNK
            pltpu.sync_copy(ii_hbm.at[pl.ds(base, CHUNK)], idx_v)
            pltpu.async_copy(table_hbm.at[idx_v], rows_v, sem).wait()
            pltpu.sync_copy(rows_v, gii_hbm.at[pl.ds(base, CHUNK)])
            pltpu.sync_copy(oi_hbm.at[pl.ds(base, CHUNK)], idx_v)
            pltpu.async_copy(table_hbm.at[idx_v], rows_v, sem).wait()
            pltpu.sync_copy(rows_v, goi_hbm.at[pl.ds(base, CHUNK)])

        return carry

    lax.fori_loop(0, CHUNK_ITERS, body, 0)


@functools.partial(
    pl.kernel,
    out_type=jax.ShapeDtypeStruct((2, N, Y_W), jnp.float32),
    mesh=_mesh,
    compiler_params=pltpu.CompilerParams(use_tc_tiling_on_sc=False),
    scratch_types=[
        pltpu.VMEM_SHARED((N, Y_W), jnp.float32),
        pltpu.VMEM((CHUNK,), jnp.int32),
        pltpu.VMEM((CHUNK, Y_W), jnp.float32),
    ],
)
def _sc_scatter(y_hbm, oi_hbm, zeros_hbm, out_hbm, acc_sh, idx_v, rows_v):
    c = lax.axis_index("c")
    s = lax.axis_index("s")
    wid = s * 2 + c
    row0 = s * ROWS_PER_TILE

    pltpu.sync_copy(
        zeros_hbm.at[pl.ds(row0, ROWS_PER_TILE)],
        acc_sh.at[pl.ds(row0, ROWS_PER_TILE)],
    )
    plsc.subcore_barrier()

    def body(j, carry):
        cid = wid + j * NW

        @pl.when(cid < NCHUNK)
        def _():
            base = cid * CHUNK
            pltpu.sync_copy(oi_hbm.at[pl.ds(base, CHUNK)], idx_v)
            pltpu.sync_copy(y_hbm.at[pl.ds(base, CHUNK)], rows_v)
            pltpu.sync_copy(rows_v, acc_sh.at[idx_v], add=True)

        return carry

    lax.fori_loop(0, CHUNK_ITERS, body, 0)
    plsc.subcore_barrier()
    pltpu.sync_copy(
        acc_sh.at[pl.ds(row0, ROWS_PER_TILE)],
        out_hbm.at[c, pl.ds(row0, ROWS_PER_TILE)],
    )


def _edge_body(gii_ref, goi_ref, w1t_ref, w2t_ref, w3t_ref, y_ref):
    gi = gii_ref[...]
    go = goi_ref[...]
    posl = gi[:, :16] - go[:, :16]
    h1 = jnp.dot(posl, w1t_ref[...], preferred_element_type=jnp.float32)
    h1 = jnp.where(h1 > 0, h1, jnp.exp(jnp.minimum(h1, 0.0)) - 1.0)
    m = jnp.dot(h1, w2t_ref[...], preferred_element_type=jnp.float32)
    m = jnp.where(m > 0, m, jnp.exp(jnp.minimum(m, 0.0)) - 1.0)
    g = jnp.dot(m, w3t_ref[...], preferred_element_type=jnp.float32)
    x = gi[:, 16:32]
    y = x[:, 0:1] * g[:, 0:C_MID]
    for cc in range(1, C_IN):
        y = y + x[:, cc:cc + 1] * g[:, cc * C_MID:(cc + 1) * C_MID]
    y_ref[:, 0:C_OUT] = y
    col = lax.broadcasted_iota(jnp.int32, (CHUNK, Y_W - C_OUT), 1)
    y_ref[:, C_OUT:Y_W] = jnp.where(col == 0, 1.0, 0.0)


def _edge_call(gii, goi, w1t, w2t, w3t):
    return pl.pallas_call(
        _edge_body,
        grid=(NCHUNK,),
        in_specs=[
            pl.BlockSpec((CHUNK, TBL_W), lambda i: (i, 0)),
            pl.BlockSpec((CHUNK, TBL_W), lambda i: (i, 0)),
            pl.BlockSpec((16, 16), lambda i: (0, 0)),
            pl.BlockSpec((16, C_MID), lambda i: (0, 0)),
            pl.BlockSpec((C_MID, C_IN * C_MID), lambda i: (0, 0)),
        ],
        out_specs=pl.BlockSpec((CHUNK, Y_W), lambda i: (i, 0)),
        out_shape=jax.ShapeDtypeStruct((E, Y_W), jnp.float32),
    )(gii, goi, w1t, w2t, w3t)


def _finish_body(p_ref, b3_ref, out_ref):
    s = p_ref[0] + p_ref[1]
    cnt = s[:, C_OUT:C_OUT + 1]
    deg = jnp.maximum(cnt, 1.0)
    y = s[:, 0:C_OUT] / deg + b3_ref[...]
    y = jnp.where(jnp.isnan(y), 0.0, y)
    y = jnp.where(y == jnp.inf, 10000.0, y)
    y = jnp.where(y == -jnp.inf, -10000.0, y)
    out_ref[...] = jnp.where(cnt > 0.5, y, 0.0)


def _finish_call(partial, b3row):
    return pl.pallas_call(
        _finish_body,
        grid=(1,),
        in_specs=[
            pl.BlockSpec((2, N, Y_W), lambda i: (0, 0, 0)),
            pl.BlockSpec((1, C_OUT), lambda i: (0, 0)),
        ],
        out_specs=pl.BlockSpec((N, C_OUT), lambda i: (0, 0)),
        out_shape=jax.ShapeDtypeStruct((N, C_OUT), jnp.float32),
    )(partial, b3row)


@jax.jit
def kernel(x_in, pos_in, batch_in, in_index, out_index, W1, W2, W3, b3):
    table = jnp.concatenate(
        [jnp.pad(pos_in, ((0, 0), (0, 13))), x_in], axis=1)
    gii, goi = _sc_gather(table, in_index, out_index)
    w1t = jnp.pad(W1.T, ((0, 13), (0, 0)))
    w2t = W2.T
    w3t = jnp.transpose(
        W3.reshape(C_OUT, C_IN, C_MID), (2, 1, 0)).reshape(C_MID, C_IN * C_MID)
    y = _edge_call(gii, goi, w1t, w2t, w3t)
    partial = _sc_scatter(y, out_index, jnp.zeros((N, Y_W), jnp.float32))
    return _finish_call(partial, b3.reshape(1, C_OUT))


# speedup vs baseline: 22.7822x; 1.3828x over previous
import functools

import jax
import jax.numpy as jnp
from jax import lax
from jax.experimental import pallas as pl
from jax.experimental.pallas import tpu as pltpu
from jax.experimental.pallas import tpu_sc as plsc

N = 10000
E = 320000
C_IN = 16
C_MID = 64
C_OUT = 64
TBL_W = 32
Y_W = 80
CHUNK = 128
NCHUNK = E // CHUNK
EBLK = 512
NEBLK = E // EBLK
NW = 32
CHUNK_ITERS = (NCHUNK + NW - 1) // NW
ROWS_PER_TILE = N // 16

_mesh = plsc.VectorSubcoreMesh(core_axis_name="c", subcore_axis_name="s")


@functools.partial(
    pl.kernel,
    out_type=(
        jax.ShapeDtypeStruct((E, TBL_W), jnp.float32),
        jax.ShapeDtypeStruct((E, TBL_W), jnp.float32),
    ),
    mesh=_mesh,
    compiler_params=pltpu.CompilerParams(use_tc_tiling_on_sc=False),
    scratch_types=[
        pltpu.VMEM((CHUNK,), jnp.int32),
        pltpu.VMEM((CHUNK, TBL_W), jnp.float32),
        pltpu.SemaphoreType.DMA,
    ],
)
def _sc_gather(table_hbm, ii_hbm, oi_hbm, gii_hbm, goi_hbm, idx_v, rows_v, sem):
    wid = lax.axis_index("s") * 2 + lax.axis_index("c")

    def body(j, carry):
        cid = wid + j * NW

        @pl.when(cid < NCHUNK)
        def _():
            base = cid * CHUNK
            pltpu.sync_copy(ii_hbm.at[pl.ds(base, CHUNK)], idx_v)
            pltpu.async_copy(table_hbm.at[idx_v], rows_v, sem).wait()
            pltpu.sync_copy(rows_v, gii_hbm.at[pl.ds(base, CHUNK)])
            pltpu.sync_copy(oi_hbm.at[pl.ds(base, CHUNK)], idx_v)
            pltpu.async_copy(table_hbm.at[idx_v], rows_v, sem).wait()
            pltpu.sync_copy(rows_v, goi_hbm.at[pl.ds(base, CHUNK)])

        return carry

    lax.fori_loop(0, CHUNK_ITERS, body, 0)


@functools.partial(
    pl.kernel,
    out_type=jax.ShapeDtypeStruct((2, N, Y_W), jnp.float32),
    mesh=_mesh,
    compiler_params=pltpu.CompilerParams(use_tc_tiling_on_sc=False),
    scratch_types=[
        pltpu.VMEM_SHARED((N, Y_W), jnp.float32),
        pltpu.VMEM((CHUNK,), jnp.int32),
        pltpu.VMEM((CHUNK, Y_W), jnp.float32),
    ],
)
def _sc_scatter(y_hbm, oi_hbm, zeros_hbm, out_hbm, acc_sh, idx_v, rows_v):
    c = lax.axis_index("c")
    s = lax.axis_index("s")
    wid = s * 2 + c
    row0 = s * ROWS_PER_TILE

    pltpu.sync_copy(
        zeros_hbm.at[pl.ds(row0, ROWS_PER_TILE)],
        acc_sh.at[pl.ds(row0, ROWS_PER_TILE)],
    )
    plsc.subcore_barrier()

    def body(j, carry):
        cid = wid + j * NW

        @pl.when(cid < NCHUNK)
        def _():
            base = cid * CHUNK
            pltpu.sync_copy(oi_hbm.at[pl.ds(base, CHUNK)], idx_v)
            pltpu.sync_copy(y_hbm.at[pl.ds(base, CHUNK)], rows_v)
            pltpu.sync_copy(rows_v, acc_sh.at[idx_v], add=True)

        return carry

    lax.fori_loop(0, CHUNK_ITERS, body, 0)
    plsc.subcore_barrier()
    pltpu.sync_copy(
        acc_sh.at[pl.ds(row0, ROWS_PER_TILE)],
        out_hbm.at[c, pl.ds(row0, ROWS_PER_TILE)],
    )


def _edge_body(gii_ref, goi_ref, w1t_ref, w2t_ref, w3t_ref, y_ref):
    gi = gii_ref[...]
    go = goi_ref[...]
    posl = gi[:, :16] - go[:, :16]
    h1 = jnp.dot(posl, w1t_ref[...], preferred_element_type=jnp.float32)
    h1 = jnp.where(h1 > 0, h1, jnp.exp(jnp.minimum(h1, 0.0)) - 1.0)
    m = jnp.dot(h1, w2t_ref[...], preferred_element_type=jnp.float32)
    m = jnp.where(m > 0, m, jnp.exp(jnp.minimum(m, 0.0)) - 1.0)
    g = jnp.dot(m, w3t_ref[...], preferred_element_type=jnp.float32)
    x = gi[:, 16:32]
    y = x[:, 0:1] * g[:, 0:C_MID]
    for cc in range(1, C_IN):
        y = y + x[:, cc:cc + 1] * g[:, cc * C_MID:(cc + 1) * C_MID]
    y_ref[:, 0:C_OUT] = y
    col = lax.broadcasted_iota(jnp.int32, (EBLK, Y_W - C_OUT), 1)
    y_ref[:, C_OUT:Y_W] = jnp.where(col == 0, 1.0, 0.0)


def _edge_call(gii, goi, w1t, w2t, w3t):
    return pl.pallas_call(
        _edge_body,
        grid=(NEBLK,),
        in_specs=[
            pl.BlockSpec((EBLK, TBL_W), lambda i: (i, 0)),
            pl.BlockSpec((EBLK, TBL_W), lambda i: (i, 0)),
            pl.BlockSpec((16, 16), lambda i: (0, 0)),
            pl.BlockSpec((16, C_MID), lambda i: (0, 0)),
            pl.BlockSpec((C_MID, C_IN * C_MID), lambda i: (0, 0)),
        ],
        out_specs=pl.BlockSpec((EBLK, Y_W), lambda i: (i, 0)),
        out_shape=jax.ShapeDtypeStruct((E, Y_W), jnp.float32),
    )(gii, goi, w1t, w2t, w3t)


def _finish_body(p_ref, b3_ref, out_ref):
    s = p_ref[0] + p_ref[1]
    cnt = s[:, C_OUT:C_OUT + 1]
    deg = jnp.maximum(cnt, 1.0)
    y = s[:, 0:C_OUT] / deg + b3_ref[...]
    y = jnp.where(jnp.isnan(y), 0.0, y)
    y = jnp.where(y == jnp.inf, 10000.0, y)
    y = jnp.where(y == -jnp.inf, -10000.0, y)
    out_ref[...] = jnp.where(cnt > 0.5, y, 0.0)


def _finish_call(partial, b3row):
    return pl.pallas_call(
        _finish_body,
        grid=(1,),
        in_specs=[
            pl.BlockSpec((2, N, Y_W), lambda i: (0, 0, 0)),
            pl.BlockSpec((1, C_OUT), lambda i: (0, 0)),
        ],
        out_specs=pl.BlockSpec((N, C_OUT), lambda i: (0, 0)),
        out_shape=jax.ShapeDtypeStruct((N, C_OUT), jnp.float32),
    )(partial, b3row)


@jax.jit
def kernel(x_in, pos_in, batch_in, in_index, out_index, W1, W2, W3, b3):
    table = jnp.concatenate(
        [jnp.pad(pos_in, ((0, 0), (0, 13))), x_in], axis=1)
    gii, goi = _sc_gather(table, in_index, out_index)
    w1t = jnp.pad(W1.T, ((0, 13), (0, 0)))
    w2t = W2.T
    w3t = jnp.transpose(
        W3.reshape(C_OUT, C_IN, C_MID), (2, 1, 0)).reshape(C_MID, C_IN * C_MID)
    y = _edge_call(gii, goi, w1t, w2t, w3t)
    partial = _sc_scatter(y, out_index, jnp.zeros((N, Y_W), jnp.float32))
    return _finish_call(partial, b3.reshape(1, C_OUT))


# speedup vs baseline: 23.5798x; 1.0350x over previous
import functools

import jax
import jax.numpy as jnp
from jax import lax
from jax.experimental import pallas as pl
from jax.experimental.pallas import tpu as pltpu
from jax.experimental.pallas import tpu_sc as plsc

N = 10000
E = 320000
C_IN = 16
C_MID = 64
C_OUT = 64
TBL_W = 32
Y_W = 80
CHUNK = 128
NCHUNK = E // CHUNK
EBLK = 1024
NEBLK = E // EBLK
NW = 32
CHUNK_ITERS = (NCHUNK + NW - 1) // NW
ROWS_PER_TILE = N // 16

_mesh = plsc.VectorSubcoreMesh(core_axis_name="c", subcore_axis_name="s")


@functools.partial(
    pl.kernel,
    out_type=(
        jax.ShapeDtypeStruct((E, TBL_W), jnp.float32),
        jax.ShapeDtypeStruct((E, TBL_W), jnp.float32),
    ),
    mesh=_mesh,
    compiler_params=pltpu.CompilerParams(use_tc_tiling_on_sc=False),
    scratch_types=[
        pltpu.VMEM((CHUNK,), jnp.int32),
        pltpu.VMEM((CHUNK, TBL_W), jnp.float32),
        pltpu.SemaphoreType.DMA,
    ],
)
def _sc_gather(table_hbm, ii_hbm, oi_hbm, gii_hbm, goi_hbm, idx_v, rows_v, sem):
    wid = lax.axis_index("s") * 2 + lax.axis_index("c")

    def body(j, carry):
        cid = wid + j * NW

        @pl.when(cid < NCHUNK)
        def _():
            base = cid * CHUNK
            pltpu.sync_copy(ii_hbm.at[pl.ds(base, CHUNK)], idx_v)
            pltpu.async_copy(table_hbm.at[idx_v], rows_v, sem).wait()
            pltpu.sync_copy(rows_v, gii_hbm.at[pl.ds(base, CHUNK)])
            pltpu.sync_copy(oi_hbm.at[pl.ds(base, CHUNK)], idx_v)
            pltpu.async_copy(table_hbm.at[idx_v], rows_v, sem).wait()
            pltpu.sync_copy(rows_v, goi_hbm.at[pl.ds(base, CHUNK)])

        return carry

    lax.fori_loop(0, CHUNK_ITERS, body, 0)


@functools.partial(
    pl.kernel,
    out_type=jax.ShapeDtypeStruct((2, N, Y_W), jnp.float32),
    mesh=_mesh,
    compiler_params=pltpu.CompilerParams(use_tc_tiling_on_sc=False),
    scratch_types=[
        pltpu.VMEM_SHARED((N, Y_W), jnp.float32),
        pltpu.VMEM((CHUNK,), jnp.int32),
        pltpu.VMEM((CHUNK, Y_W), jnp.float32),
    ],
)
def _sc_scatter(y_hbm, oi_hbm, zeros_hbm, out_hbm, acc_sh, idx_v, rows_v):
    c = lax.axis_index("c")
    s = lax.axis_index("s")
    wid = s * 2 + c
    row0 = s * ROWS_PER_TILE

    pltpu.sync_copy(
        zeros_hbm.at[pl.ds(row0, ROWS_PER_TILE)],
        acc_sh.at[pl.ds(row0, ROWS_PER_TILE)],
    )
    plsc.subcore_barrier()

    def body(j, carry):
        cid = wid + j * NW

        @pl.when(cid < NCHUNK)
        def _():
            base = cid * CHUNK
            pltpu.sync_copy(oi_hbm.at[pl.ds(base, CHUNK)], idx_v)
            pltpu.sync_copy(y_hbm.at[pl.ds(base, CHUNK)], rows_v)
            pltpu.sync_copy(rows_v, acc_sh.at[idx_v], add=True)

        return carry

    lax.fori_loop(0, CHUNK_ITERS, body, 0)
    plsc.subcore_barrier()
    pltpu.sync_copy(
        acc_sh.at[pl.ds(row0, ROWS_PER_TILE)],
        out_hbm.at[c, pl.ds(row0, ROWS_PER_TILE)],
    )


def _edge_body(gii_ref, goi_ref, w1t_ref, w2t_ref, w3t_ref, y_ref):
    gi = gii_ref[...]
    go = goi_ref[...]
    posl = gi[:, :16] - go[:, :16]
    h1 = jnp.dot(posl, w1t_ref[...], preferred_element_type=jnp.float32)
    h1 = jnp.where(h1 > 0, h1, jnp.exp(jnp.minimum(h1, 0.0)) - 1.0)
    m = jnp.dot(h1, w2t_ref[...], preferred_element_type=jnp.float32)
    m = jnp.where(m > 0, m, jnp.exp(jnp.minimum(m, 0.0)) - 1.0)
    g = jnp.dot(m.astype(jnp.bfloat16), w3t_ref[...],
                preferred_element_type=jnp.float32)
    x = gi[:, 16:32]
    y = x[:, 0:1] * g[:, 0:C_MID]
    for cc in range(1, C_IN):
        y = y + x[:, cc:cc + 1] * g[:, cc * C_MID:(cc + 1) * C_MID]
    y_ref[:, 0:C_OUT] = y
    col = lax.broadcasted_iota(jnp.int32, (EBLK, Y_W - C_OUT), 1)
    y_ref[:, C_OUT:Y_W] = jnp.where(col == 0, 1.0, 0.0)


def _edge_call(gii, goi, w1t, w2t, w3t):
    return pl.pallas_call(
        _edge_body,
        grid=(NEBLK,),
        in_specs=[
            pl.BlockSpec((EBLK, TBL_W), lambda i: (i, 0)),
            pl.BlockSpec((EBLK, TBL_W), lambda i: (i, 0)),
            pl.BlockSpec((16, 16), lambda i: (0, 0)),
            pl.BlockSpec((16, C_MID), lambda i: (0, 0)),
            pl.BlockSpec((C_MID, C_IN * C_MID), lambda i: (0, 0)),
        ],
        out_specs=pl.BlockSpec((EBLK, Y_W), lambda i: (i, 0)),
        out_shape=jax.ShapeDtypeStruct((E, Y_W), jnp.float32),
    )(gii, goi, w1t, w2t, w3t.astype(jnp.bfloat16))


def _finish_body(p_ref, b3_ref, out_ref):
    s = p_ref[0] + p_ref[1]
    cnt = s[:, C_OUT:C_OUT + 1]
    deg = jnp.maximum(cnt, 1.0)
    y = s[:, 0:C_OUT] / deg + b3_ref[...]
    y = jnp.where(jnp.isnan(y), 0.0, y)
    y = jnp.where(y == jnp.inf, 10000.0, y)
    y = jnp.where(y == -jnp.inf, -10000.0, y)
    out_ref[...] = jnp.where(cnt > 0.5, y, 0.0)


def _finish_call(partial, b3row):
    return pl.pallas_call(
        _finish_body,
        grid=(1,),
        in_specs=[
            pl.BlockSpec((2, N, Y_W), lambda i: (0, 0, 0)),
            pl.BlockSpec((1, C_OUT), lambda i: (0, 0)),
        ],
        out_specs=pl.BlockSpec((N, C_OUT), lambda i: (0, 0)),
        out_shape=jax.ShapeDtypeStruct((N, C_OUT), jnp.float32),
    )(partial, b3row)


@jax.jit
def kernel(x_in, pos_in, batch_in, in_index, out_index, W1, W2, W3, b3):
    table = jnp.concatenate(
        [jnp.pad(pos_in, ((0, 0), (0, 13))), x_in], axis=1)
    gii, goi = _sc_gather(table, in_index, out_index)
    w1t = jnp.pad(W1.T, ((0, 13), (0, 0)))
    w2t = W2.T
    w3t = jnp.transpose(
        W3.reshape(C_OUT, C_IN, C_MID), (2, 1, 0)).reshape(C_MID, C_IN * C_MID)
    y = _edge_call(gii, goi, w1t, w2t, w3t)
    partial = _sc_scatter(y, out_index, jnp.zeros((N, Y_W), jnp.float32))
    return _finish_call(partial, b3.reshape(1, C_OUT))


# speedup vs baseline: 27.6640x; 1.1732x over previous
import functools

import jax
import jax.numpy as jnp
from jax import lax
from jax.experimental import pallas as pl
from jax.experimental.pallas import tpu as pltpu
from jax.experimental.pallas import tpu_sc as plsc

N = 10000
E = 320000
C_IN = 16
C_MID = 64
C_OUT = 64
TBL_W = 32
Y_W = 80
CHUNK = 128
NCHUNK = E // CHUNK
EBLK = 1024
NEBLK = E // EBLK
NW = 32
CHUNK_ITERS = (NCHUNK + NW - 1) // NW
ROWS_PER_TILE = N // 16

_mesh = plsc.VectorSubcoreMesh(core_axis_name="c", subcore_axis_name="s")


@functools.partial(
    pl.kernel,
    out_type=(
        jax.ShapeDtypeStruct((E, TBL_W), jnp.float32),
        jax.ShapeDtypeStruct((E, TBL_W), jnp.float32),
    ),
    mesh=_mesh,
    compiler_params=pltpu.CompilerParams(use_tc_tiling_on_sc=False),
    scratch_types=[
        pltpu.VMEM((CHUNK,), jnp.int32),
        pltpu.VMEM((CHUNK, TBL_W), jnp.float32),
        pltpu.SemaphoreType.DMA,
    ],
)
def _sc_gather(table_hbm, ii_hbm, oi_hbm, gii_hbm, goi_hbm, idx_v, rows_v, sem):
    wid = lax.axis_index("s") * 2 + lax.axis_index("c")

    def body(j, carry):
        cid = wid + j * NW

        @pl.when(cid < NCHUNK)
        def _():
            base = cid * CHUNK
            pltpu.sync_copy(ii_hbm.at[pl.ds(base, CHUNK)], idx_v)
            pltpu.async_copy(table_hbm.at[idx_v], rows_v, sem).wait()
            pltpu.sync_copy(rows_v, gii_hbm.at[pl.ds(base, CHUNK)])
            pltpu.sync_copy(oi_hbm.at[pl.ds(base, CHUNK)], idx_v)
            pltpu.async_copy(table_hbm.at[idx_v], rows_v, sem).wait()
            pltpu.sync_copy(rows_v, goi_hbm.at[pl.ds(base, CHUNK)])

        return carry

    lax.fori_loop(0, CHUNK_ITERS, body, 0)


@functools.partial(
    pl.kernel,
    out_type=jax.ShapeDtypeStruct((2, N, Y_W), jnp.float32),
    mesh=_mesh,
    compiler_params=pltpu.CompilerParams(use_tc_tiling_on_sc=False),
    scratch_types=[
        pltpu.VMEM_SHARED((N, Y_W), jnp.float32),
        pltpu.VMEM((CHUNK,), jnp.int32),
        pltpu.VMEM((CHUNK, Y_W), jnp.float32),
    ],
)
def _sc_scatter(y_hbm, oi_hbm, zeros_hbm, out_hbm, acc_sh, idx_v, rows_v):
    c = lax.axis_index("c")
    s = lax.axis_index("s")
    wid = s * 2 + c
    row0 = s * ROWS_PER_TILE

    pltpu.sync_copy(
        zeros_hbm.at[pl.ds(row0, ROWS_PER_TILE)],
        acc_sh.at[pl.ds(row0, ROWS_PER_TILE)],
    )
    plsc.subcore_barrier()

    def body(j, carry):
        cid = wid + j * NW

        @pl.when(cid < NCHUNK)
        def _():
            base = cid * CHUNK
            pltpu.sync_copy(oi_hbm.at[pl.ds(base, CHUNK)], idx_v)
            pltpu.sync_copy(y_hbm.at[pl.ds(base, CHUNK)], rows_v)
            pltpu.sync_copy(rows_v, acc_sh.at[idx_v], add=True)

        return carry

    lax.fori_loop(0, CHUNK_ITERS, body, 0)
    plsc.subcore_barrier()
    pltpu.sync_copy(
        acc_sh.at[pl.ds(row0, ROWS_PER_TILE)],
        out_hbm.at[c, pl.ds(row0, ROWS_PER_TILE)],
    )


def _edge_body(gii_ref, goi_ref, w1t_ref, w2t_ref, w3t_ref, brd_ref, red_ref,
               y_ref):
    gi = gii_ref[...]
    go = goi_ref[...]
    posl = gi[:, :16] - go[:, :16]
    h1 = jnp.dot(posl, w1t_ref[...], preferred_element_type=jnp.float32)
    h1 = jnp.where(h1 > 0, h1, jnp.exp(jnp.minimum(h1, 0.0)) - 1.0)
    m = jnp.dot(h1, w2t_ref[...], preferred_element_type=jnp.float32)
    m = jnp.where(m > 0, m, jnp.exp(jnp.minimum(m, 0.0)) - 1.0)
    g = jnp.dot(m, w3t_ref[...], preferred_element_type=jnp.float32)
    x = gi[:, 16:32]
    xb = jnp.dot(x, brd_ref[...], preferred_element_type=jnp.float32)
    y = jnp.dot(xb * g, red_ref[...], preferred_element_type=jnp.float32)
    y_ref[:, 0:C_OUT] = y
    col = lax.broadcasted_iota(jnp.int32, (EBLK, Y_W - C_OUT), 1)
    y_ref[:, C_OUT:Y_W] = jnp.where(col == 0, 1.0, 0.0)


def _edge_call(gii, goi, w1t, w2t, w3t, brd, red):
    return pl.pallas_call(
        _edge_body,
        grid=(NEBLK,),
        in_specs=[
            pl.BlockSpec((EBLK, TBL_W), lambda i: (i, 0)),
            pl.BlockSpec((EBLK, TBL_W), lambda i: (i, 0)),
            pl.BlockSpec((16, 16), lambda i: (0, 0)),
            pl.BlockSpec((16, C_MID), lambda i: (0, 0)),
            pl.BlockSpec((C_MID, C_IN * C_MID), lambda i: (0, 0)),
            pl.BlockSpec((C_IN, C_IN * C_MID), lambda i: (0, 0)),
            pl.BlockSpec((C_IN * C_MID, C_OUT), lambda i: (0, 0)),
        ],
        out_specs=pl.BlockSpec((EBLK, Y_W), lambda i: (i, 0)),
        out_shape=jax.ShapeDtypeStruct((E, Y_W), jnp.float32),
    )(gii, goi, w1t, w2t, w3t, brd, red)


def _finish_body(p_ref, b3_ref, out_ref):
    s = p_ref[0] + p_ref[1]
    cnt = s[:, C_OUT:C_OUT + 1]
    deg = jnp.maximum(cnt, 1.0)
    y = s[:, 0:C_OUT] / deg + b3_ref[...]
    y = jnp.where(jnp.isnan(y), 0.0, y)
    y = jnp.where(y == jnp.inf, 10000.0, y)
    y = jnp.where(y == -jnp.inf, -10000.0, y)
    out_ref[...] = jnp.where(cnt > 0.5, y, 0.0)


def _finish_call(partial, b3row):
    return pl.pallas_call(
        _finish_body,
        grid=(1,),
        in_specs=[
            pl.BlockSpec((2, N, Y_W), lambda i: (0, 0, 0)),
            pl.BlockSpec((1, C_OUT), lambda i: (0, 0)),
        ],
        out_specs=pl.BlockSpec((N, C_OUT), lambda i: (0, 0)),
        out_shape=jax.ShapeDtypeStruct((N, C_OUT), jnp.float32),
    )(partial, b3row)


@jax.jit
def kernel(x_in, pos_in, batch_in, in_index, out_index, W1, W2, W3, b3):
    table = jnp.concatenate(
        [jnp.pad(pos_in, ((0, 0), (0, 13))), x_in], axis=1)
    gii, goi = _sc_gather(table, in_index, out_index)
    w1t = jnp.pad(W1.T, ((0, 13), (0, 0)))
    w2t = W2.T
    w3t = jnp.transpose(
        W3.reshape(C_OUT, C_IN, C_MID), (2, 1, 0)).reshape(C_MID, C_IN * C_MID)
    ids = jnp.arange(C_IN * C_MID, dtype=jnp.int32)
    brd = (jnp.arange(C_IN, dtype=jnp.int32)[:, None]
           == ids[None, :] // C_MID).astype(jnp.float32)
    red = (ids[:, None] % C_MID
           == jnp.arange(C_OUT, dtype=jnp.int32)[None, :]).astype(jnp.float32)
    y = _edge_call(gii, goi, w1t, w2t, w3t, brd, red)
    partial = _sc_scatter(y, out_index, jnp.zeros((N, Y_W), jnp.float32))
    return _finish_call(partial, b3.reshape(1, C_OUT))


# speedup vs baseline: 27.6962x; 1.0012x over previous
import functools

import jax
import jax.numpy as jnp
from jax import lax
from jax.experimental import pallas as pl
from jax.experimental.pallas import tpu as pltpu
from jax.experimental.pallas import tpu_sc as plsc

N = 10000
E = 320000
C_IN = 16
C_MID = 64
C_OUT = 64
TBL_W = 32
Y_W = 80
CHUNK = 128
NCHUNK = E // CHUNK
EBLK = 1024
NEBLK = E // EBLK
NW = 32
CHUNK_ITERS = (NCHUNK + NW - 1) // NW
ROWS_PER_TILE = N // 16

_mesh = plsc.VectorSubcoreMesh(core_axis_name="c", subcore_axis_name="s")


@functools.partial(
    pl.kernel,
    out_type=(
        jax.ShapeDtypeStruct((E, TBL_W), jnp.float32),
        jax.ShapeDtypeStruct((E, TBL_W), jnp.float32),
    ),
    mesh=_mesh,
    compiler_params=pltpu.CompilerParams(use_tc_tiling_on_sc=False),
    scratch_types=[
        pltpu.VMEM((CHUNK,), jnp.int32),
        pltpu.VMEM((CHUNK, TBL_W), jnp.float32),
        pltpu.SemaphoreType.DMA,
    ],
)
def _sc_gather(table_hbm, ii_hbm, oi_hbm, gii_hbm, goi_hbm, idx_v, rows_v, sem):
    wid = lax.axis_index("s") * 2 + lax.axis_index("c")

    def body(j, carry):
        cid = wid + j * NW

        @pl.when(cid < NCHUNK)
        def _():
            base = cid * CHUNK
            pltpu.sync_copy(ii_hbm.at[pl.ds(base, CHUNK)], idx_v)
            pltpu.async_copy(table_hbm.at[idx_v], rows_v, sem).wait()
            pltpu.sync_copy(rows_v, gii_hbm.at[pl.ds(base, CHUNK)])
            pltpu.sync_copy(oi_hbm.at[pl.ds(base, CHUNK)], idx_v)
            pltpu.async_copy(table_hbm.at[idx_v], rows_v, sem).wait()
            pltpu.sync_copy(rows_v, goi_hbm.at[pl.ds(base, CHUNK)])

        return carry

    lax.fori_loop(0, CHUNK_ITERS, body, 0)


@functools.partial(
    pl.kernel,
    out_type=jax.ShapeDtypeStruct((2, N, Y_W), jnp.float32),
    mesh=_mesh,
    compiler_params=pltpu.CompilerParams(use_tc_tiling_on_sc=False),
    scratch_types=[
        pltpu.VMEM_SHARED((N, Y_W), jnp.float32),
        pltpu.VMEM((CHUNK,), jnp.int32),
        pltpu.VMEM((CHUNK, Y_W), jnp.float32),
    ],
)
def _sc_scatter(y_hbm, oi_hbm, zeros_hbm, out_hbm, acc_sh, idx_v, rows_v):
    c = lax.axis_index("c")
    s = lax.axis_index("s")
    wid = s * 2 + c
    row0 = s * ROWS_PER_TILE

    pltpu.sync_copy(
        zeros_hbm.at[pl.ds(row0, ROWS_PER_TILE)],
        acc_sh.at[pl.ds(row0, ROWS_PER_TILE)],
    )
    plsc.subcore_barrier()

    def body(j, carry):
        cid = wid + j * NW

        @pl.when(cid < NCHUNK)
        def _():
            base = cid * CHUNK
            pltpu.sync_copy(oi_hbm.at[pl.ds(base, CHUNK)], idx_v)
            pltpu.sync_copy(y_hbm.at[pl.ds(base, CHUNK)], rows_v)
            pltpu.sync_copy(rows_v, acc_sh.at[idx_v], add=True)

        return carry

    lax.fori_loop(0, CHUNK_ITERS, body, 0)
    plsc.subcore_barrier()
    pltpu.sync_copy(
        acc_sh.at[pl.ds(row0, ROWS_PER_TILE)],
        out_hbm.at[c, pl.ds(row0, ROWS_PER_TILE)],
    )


def _edge_body(gii_ref, goi_ref, w1t_ref, w2t_ref, w3t_ref, brd_ref,
               red_ref, y_ref):
    gi = gii_ref[...]
    go = goi_ref[...]
    posl = gi[:, :16] - go[:, :16]
    h1 = jnp.dot(posl, w1t_ref[...], preferred_element_type=jnp.float32)
    h1 = jnp.where(h1 > 0, h1, jnp.exp(jnp.minimum(h1, 0.0)) - 1.0)
    m = jnp.dot(h1, w2t_ref[...], preferred_element_type=jnp.float32)
    m = jnp.where(m > 0, m, jnp.exp(jnp.minimum(m, 0.0)) - 1.0)
    g = jnp.dot(m, w3t_ref[...], preferred_element_type=jnp.float32)
    xb = jnp.dot(gi, brd_ref[...], preferred_element_type=jnp.float32)
    y = jnp.dot(xb * g, red_ref[...], preferred_element_type=jnp.float32)
    y_ref[:, 0:C_OUT] = y
    col = lax.broadcasted_iota(jnp.int32, (EBLK, Y_W - C_OUT), 1)
    y_ref[:, C_OUT:Y_W] = jnp.where(col == 0, 1.0, 0.0)


def _edge_call(gii, goi, w1t, w2t, w3t, brd, red):
    return pl.pallas_call(
        _edge_body,
        grid=(NEBLK,),
        in_specs=[
            pl.BlockSpec((EBLK, TBL_W), lambda i: (i, 0)),
            pl.BlockSpec((EBLK, TBL_W), lambda i: (i, 0)),
            pl.BlockSpec((16, 16), lambda i: (0, 0)),
            pl.BlockSpec((16, C_MID), lambda i: (0, 0)),
            pl.BlockSpec((C_MID, C_IN * C_MID), lambda i: (0, 0)),
            pl.BlockSpec((TBL_W, C_IN * C_MID), lambda i: (0, 0)),
            pl.BlockSpec((C_IN * C_MID, C_OUT), lambda i: (0, 0)),
        ],
        out_specs=pl.BlockSpec((EBLK, Y_W), lambda i: (i, 0)),
        out_shape=jax.ShapeDtypeStruct((E, Y_W), jnp.float32),
    )(gii, goi, w1t, w2t, w3t, brd, red)


def _finish_body(p_ref, b3_ref, out_ref):
    s = p_ref[0] + p_ref[1]
    cnt = s[:, C_OUT:C_OUT + 1]
    deg = jnp.maximum(cnt, 1.0)
    y = s[:, 0:C_OUT] / deg + b3_ref[...]
    y = jnp.where(jnp.isnan(y), 0.0, y)
    y = jnp.where(y == jnp.inf, 10000.0, y)
    y = jnp.where(y == -jnp.inf, -10000.0, y)
    out_ref[...] = jnp.where(cnt > 0.5, y, 0.0)


def _finish_call(partial, b3row):
    return pl.pallas_call(
        _finish_body,
        grid=(1,),
        in_specs=[
            pl.BlockSpec((2, N, Y_W), lambda i: (0, 0, 0)),
            pl.BlockSpec((1, C_OUT), lambda i: (0, 0)),
        ],
        out_specs=pl.BlockSpec((N, C_OUT), lambda i: (0, 0)),
        out_shape=jax.ShapeDtypeStruct((N, C_OUT), jnp.float32),
    )(partial, b3row)


@jax.jit
def kernel(x_in, pos_in, batch_in, in_index, out_index, W1, W2, W3, b3):
    table = jnp.concatenate(
        [jnp.pad(pos_in, ((0, 0), (0, 13))), x_in], axis=1)
    gii, goi = _sc_gather(table, in_index, out_index)
    w1t = jnp.pad(W1.T, ((0, 13), (0, 0)))
    w2t = W2.T
    w3t = jnp.transpose(
        W3.reshape(C_OUT, C_IN, C_MID), (2, 1, 0)).reshape(C_MID, C_IN * C_MID)
    ids = jnp.arange(C_IN * C_MID, dtype=jnp.int32)
    brd = (jnp.arange(TBL_W, dtype=jnp.int32)[:, None] - 16
           == ids[None, :] // C_MID).astype(jnp.float32)
    red = (ids[:, None] % C_MID
           == jnp.arange(C_OUT, dtype=jnp.int32)[None, :]).astype(jnp.float32)
    y = _edge_call(gii, goi, w1t, w2t, w3t, brd, red)
    partial = _sc_scatter(y, out_index, jnp.zeros((N, Y_W), jnp.float32))
    return _finish_call(partial, b3.reshape(1, C_OUT))
